# speedup vs baseline: 2.4022x; 2.4022x over previous
import functools

import jax
import jax.numpy as jnp
from jax import lax
from jax.experimental import pallas as pl
from jax.experimental.pallas import tpu as pltpu
from jax.experimental.pallas import tpu_sc as plsc

B, L, D = 4096, 50, 128
N = B * L
NF = 5
NC, NS = 2, 16
NW = NC * NS
TPW = N // NW
C = 128
NCHUNK = TPW // C


def _sc_body(label_h, x_h, y_h, w_h, h_h,
             lt_h, xt_h, yt_h, wt_h, ht_h,
             out_h,
             i0, i1, i2, i3, i4,
             r0, r1, r2, r3, r4,
             gsem):
  wid = lax.axis_index("s") * NC + lax.axis_index("c")
  base = wid * TPW

  idx_refs = (i0, i1, i2, i3, i4)
  row_refs = (r0, r1, r2, r3, r4)
  idx_hbms = (label_h, x_h, y_h, w_h, h_h)
  tab_hbms = (lt_h, xt_h, yt_h, wt_h, ht_h)

  def chunk(ci, carry):
    tb = base + ci * C
    for f in range(NF):
      pltpu.sync_copy(idx_hbms[f].at[pl.ds(tb, C)], idx_refs[f])
    descs = [
        pltpu.async_copy(tab_hbms[f].at[idx_refs[f]], row_refs[f], gsem)
        for f in range(NF)
    ]
    for d in descs:
      d.wait()
    for f in range(NF):
      pltpu.sync_copy(row_refs[f],
                      out_h.at[pl.ds(tb, C), pl.ds(f * D, D)])
    return carry

  lax.fori_loop(0, NCHUNK, chunk, 0)


@jax.jit
def kernel(label, x, y, w, h, label_table, x_table, y_table, w_table, h_table):
  idx = [a.reshape(N).astype(jnp.int32) for a in (label, x, y, w, h)]
  mesh = plsc.VectorSubcoreMesh(core_axis_name="c", subcore_axis_name="s",
                                num_cores=NC, num_subcores=NS)
  run = pl.kernel(
      _sc_body,
      out_type=jax.ShapeDtypeStruct((N, NF * D), jnp.float32),
      mesh=mesh,
      scratch_types=(
          [pltpu.VMEM((C,), jnp.int32) for _ in range(NF)]
          + [pltpu.VMEM((C, D), jnp.float32) for _ in range(NF)]
          + [pltpu.SemaphoreType.DMA]
      ),
  )
  out = run(*idx, label_table, x_table, y_table, w_table, h_table)
  return out.reshape(B, L, NF * D)
